# baseline (speedup 1.0000x reference)
import jax
import jax.numpy as jnp
from jax.experimental import pallas as pl
from jax.experimental.pallas import tpu as pltpu

_B, _S, _H = 4, 256, 128
_E = 2 * _H
_MAX_LR = 0.01
_NB = 4


def _sig(x):
    return jax.nn.sigmoid(x)


def _dot(a, b):
    return jax.lax.dot_general(a, b, (((1,), (0,)), ((), ())),
                               preferred_element_type=jnp.float32)


def _dot_t(a, b):
    return jax.lax.dot_general(a, b, (((1,), (1,)), ((), ())),
                               preferred_element_type=jnp.float32)


def _outer(a, b):
    return jax.lax.dot_general(a.astype(jnp.bfloat16), b.astype(jnp.bfloat16),
                               (((0,), (0,)), ((), ())),
                               preferred_element_type=jnp.float32)


def _l2n(x):
    n = jnp.sqrt(jnp.sum(x * x, axis=-1, keepdims=True))
    return x / jnp.maximum(n, 1e-12)


def _mem_kernel(x_ref, wbig_ref, bbig_ref,
                w1_0_ref, b1_0_ref, w2_0_ref, b2_0_ref,
                w1_1_ref, b1_1_ref, w2_1_ref, b2_1_ref,
                mw1_0_ref, mb1_0_ref, mw2_0_ref, mb2_0_ref,
                mw1_1_ref, mb1_1_ref, mw2_1_ref, mb2_1_ref,
                o_ref,
                kr, vr, qr, scr, pq0, pk0,
                pw1b, pw2, ww,
                mw1, mb1, mw2, mb2):
    w1_refs = (w1_0_ref, w1_1_ref)
    b1_refs = (b1_0_ref, b1_1_ref)
    w2_refs = (w2_0_ref, w2_1_ref)
    b2_refs = (b2_0_ref, b2_1_ref)
    mw1_refs = (mw1_0_ref, mw1_1_ref)
    mb1_refs = (mb1_0_ref, mb1_1_ref)
    mw2_refs = (mw2_0_ref, mw2_1_ref)
    mb2_refs = (mb2_0_ref, mb2_1_ref)

    p00 = (w1_0_ref[...], b1_0_ref[...], w2_0_ref[...], b2_0_ref[...])
    p01 = (w1_1_ref[...], b1_1_ref[...], w2_1_ref[...], b2_1_ref[...])

    for b in range(_NB):
        xb = x_ref[b]
        proj = _dot(xb, wbig_ref[...]) + bbig_ref[...]
        qp = proj[:, 0:_H]
        kp = proj[:, _H:2 * _H]
        vp = proj[:, 2 * _H:3 * _H]
        gp = proj[:, 3 * _H:4 * _H]
        qn = _l2n(qp * _sig(qp))
        kn = _l2n(kp * _sig(kp))
        vn = vp * _sig(vp)
        sc = _sig(gp)
        cs = slice(b * _H, (b + 1) * _H)
        ce = slice(b * _E, (b + 1) * _E)
        qr[:, :, cs] = qn.reshape(_S, 1, _H)
        kr[0:_S, :, cs] = kn.reshape(_S, 1, _H)
        kr[_S:_S + 1, :, cs] = jnp.zeros((1, 1, _H), jnp.float32)
        vr[:, :, cs] = vn.reshape(_S, 1, _H)
        scr[:, :, cs] = sc.reshape(_S, 1, _H)
        pq0[:, :, ce] = _dot(qn, p00[0]).reshape(_S, 1, _E)
        pk0[0:_S, :, ce] = _dot(kn, p00[0]).reshape(_S, 1, _E)
        pk0[_S:_S + 1, :, ce] = jnp.zeros((1, 1, _E), jnp.float32)
        pw1b[b, 0] = w1_refs[1][...]
        for d in range(2):
            pw2[b, d] = w2_refs[d][...]
            mw1[b, d] = mw1_refs[d][...]
            mb1[b, d] = mb1_refs[d][...]
            mw2[b, d] = mw2_refs[d][...]
            mb2[b, d] = mb2_refs[d][...]

    wwinit = _dot(p00[2], p01[0]).astype(jnp.bfloat16)
    for b in range(_NB):
        ww[b] = wwinit

    krow0 = kr[0]
    init = []
    for b in range(_NB):
        cs = slice(b * _H, (b + 1) * _H)
        kt = krow0[:, cs]
        a1 = _dot(kt, p00[0]) + p00[1]
        sg1 = _sig(a1)
        s1 = a1 * sg1
        dsl1 = sg1 * (1.0 + a1 * (1.0 - sg1))
        h1 = kt + _dot(s1, p00[2]) + p00[3]
        a2 = _dot(h1, p01[0]) + p01[1]
        sg2 = _sig(a2)
        s2 = a2 * sg2
        dsl2 = sg2 * (1.0 + a2 * (1.0 - sg2))
        h2 = h1 + _dot(s2, p01[2]) + p01[3]
        init.append((s1, dsl1, h1, s2, dsl2, h2))

    def substep(t, carry):
        krow1 = kr[t + 1]
        vrow = vr[t]
        qrow = qr[t]
        srow = scr[t]
        krow = kr[t]
        pqrow = pq0[t]
        pkrow = pk0[t + 1]
        new_carry = []
        for b in range(_NB):
            cs = slice(b * _H, (b + 1) * _H)
            ce = slice(b * _E, (b + 1) * _E)
            s1, dsl1, h1, s2, dsl2, h2 = carry[b]
            kt = krow[:, cs]
            vt = vrow[:, cs]
            th = srow[:, b * _H:b * _H + 1] * _MAX_LR
            al = srow[:, b * _H + 1:b * _H + 2]
            et = srow[:, b * _H + 2:b * _H + 3]

            w1b = pw1b[b, 0]
            w2a = pw2[b, 0]
            w2b = pw2[b, 1]
            wwb = ww[b]
            cm_w1a = mw1[b, 0]
            cm_b1a = mb1[b, 0]
            cm_w2a = mw2[b, 0]
            cm_b2a = mb2[b, 0]
            cm_w1b = mw1[b, 1]
            cm_b1b = mb1[b, 1]
            cm_w2b = mw2[b, 1]
            cm_b2b = mb2[b, 1]

            x2 = jnp.concatenate([qrow[:, cs], krow1[:, cs]], axis=0)
            p2 = jnp.concatenate([pqrow[:, ce], pkrow[:, ce]], axis=0)
            mx = _dot(x2, cm_w1a)
            rr = jnp.sum(x2 * kt, axis=1, keepdims=True)

            err = (2.0 / _H) * (h2 - vt)

            ds2 = _dot_t(err, w2b)
            ew2a = _dot_t(err, w2a)
            da2 = ds2 * dsl2
            dh1 = err + _dot_t(da2, w1b)
            ds1 = ew2a + jax.lax.dot_general(
                da2.astype(jnp.bfloat16), wwb, (((1,), (1,)), ((), ())),
                preferred_element_type=jnp.float32)
            da1 = ds1 * dsl1

            oma = 1.0 - al
            c1 = (oma * (p2 + p00[1]) + et * (mx + cm_b1a)
                  - th * ((rr + 1.0) * da1))
            sgc1 = _sig(c1)
            sc1 = c1 * sgc1
            u1p = _dot(sc1, p00[2])
            u1m = _dot(sc1, cm_w2a)
            r1 = jnp.sum(sc1 * s1, axis=1, keepdims=True)
            u1 = (x2 + oma * (u1p + p00[3]) + et * (u1m + cm_b2a)
                  - th * ((r1 + 1.0) * dh1))

            g_w2b = _outer(s2, err)
            g_w1b = _outer(h1, da2)
            g_w2a = _outer(s1, dh1)
            g_w1a = _outer(kt, da1)

            nm_w1a = et * cm_w1a - th * g_w1a
            nm_b1a = et * cm_b1a - th * da1
            nm_w2a = et * cm_w2a - th * g_w2a
            nm_b2a = et * cm_b2a - th * dh1
            nm_w1b = et * cm_w1b - th * g_w1b
            nm_b1b = et * cm_b1b - th * da2
            nm_w2b = et * cm_w2b - th * g_w2b
            nm_b2b = et * cm_b2b - th * err
            np_w2a = p00[2] * oma + nm_w2a
            np_w1b = p01[0] * oma + nm_w1b
            np_b1b = p01[1] * oma + nm_b1b
            np_w2b = p01[2] * oma + nm_w2b
            np_b2b = p01[3] * oma + nm_b2b

            c2 = _dot(u1, np_w1b) + np_b1b
            sgc2 = _sig(c2)
            sc2 = c2 * sgc2
            u2 = u1 + _dot(sc2, np_w2b) + np_b2b
            mw1[b, 0] = nm_w1a
            mb1[b, 0] = nm_b1a
            mw2[b, 0] = nm_w2a
            mb2[b, 0] = nm_b2a
            mw1[b, 1] = nm_w1b
            mb1[b, 1] = nm_b1b
            mw2[b, 1] = nm_w2b
            mb2[b, 1] = nm_b2b
            pw2[b, 0] = np_w2a
            pw1b[b, 0] = np_w1b
            pw2[b, 1] = np_w2b
            ww[b] = jax.lax.dot_general(
                np_w2a.astype(jnp.bfloat16), np_w1b.astype(jnp.bfloat16),
                (((1,), (0,)), ((), ())),
                preferred_element_type=jnp.float32,
                ).astype(jnp.bfloat16)

            dslc1 = sgc1 * (1.0 + c1 * (1.0 - sgc1))
            dslc2 = sgc2 * (1.0 + c2 * (1.0 - sgc2))

            o_ref[pl.ds(t, 1), :, cs] = u2[0:1, :].reshape(1, 1, _H)
            new_carry.append((sc1[1:2, :], dslc1[1:2, :], u1[1:2, :],
                              sc2[1:2, :], dslc2[1:2, :], u2[1:2, :]))
        return tuple(new_carry)

    def step2(i, carry):
        t0 = i * 2
        carry = substep(t0, carry)
        carry = substep(t0 + 1, carry)
        return carry

    jax.lax.fori_loop(0, _S // 2, step2, tuple(init))


def kernel(x, Wq, Wk, Wv, w_lr, b_lr, w_fg, b_fg, w_mo, b_mo,
           w1_0, b1_0, w2_0, b2_0, w1_1, b1_1, w2_1, b2_1,
           m_w1_0, m_b1_0, m_w2_0, m_b2_0, m_w1_1, m_b1_1, m_w2_1, m_b2_1):
    lr3 = jnp.concatenate([w_lr, w_fg, w_mo], axis=1)
    lrpad = jnp.pad(lr3, ((0, 0), (0, _H - 3)))
    wbig = jnp.concatenate([Wq, Wk, Wv, lrpad], axis=1)
    bbig = jnp.zeros((1, 4 * _H), jnp.float32)
    bbig = bbig.at[0, 3 * _H].set(b_lr[0])
    bbig = bbig.at[0, 3 * _H + 1].set(b_fg[0])
    bbig = bbig.at[0, 3 * _H + 2].set(b_mo[0])

    r = lambda a: a.reshape(1, -1)

    def wspec(shape):
        return pl.BlockSpec(shape, lambda i: (0,) * len(shape))

    out = pl.pallas_call(
        _mem_kernel,
        grid=(1,),
        in_specs=[
            pl.BlockSpec((_NB, _S, _H), lambda i: (0, 0, 0)),
            wspec((_H, 4 * _H)), wspec((1, 4 * _H)),
            wspec((_H, _E)), wspec((1, _E)), wspec((_E, _H)), wspec((1, _H)),
            wspec((_H, _E)), wspec((1, _E)), wspec((_E, _H)), wspec((1, _H)),
            wspec((_H, _E)), wspec((1, _E)), wspec((_E, _H)), wspec((1, _H)),
            wspec((_H, _E)), wspec((1, _E)), wspec((_E, _H)), wspec((1, _H)),
        ],
        out_specs=pl.BlockSpec((_S, 1, _NB * _H), lambda i: (0, 0, 0)),
        out_shape=jax.ShapeDtypeStruct((_S, 1, _B * _H), jnp.float32),
        scratch_shapes=[
            pltpu.VMEM((_S + 1, 1, _NB * _H), jnp.float32),
            pltpu.VMEM((_S, 1, _NB * _H), jnp.float32),
            pltpu.VMEM((_S, 1, _NB * _H), jnp.float32),
            pltpu.VMEM((_S, 1, _NB * _H), jnp.float32),
            pltpu.VMEM((_S, 1, _NB * _E), jnp.float32),
            pltpu.VMEM((_S + 1, 1, _NB * _E), jnp.float32),
            pltpu.VMEM((_NB, 1, _H, _E), jnp.float32),
            pltpu.VMEM((_NB, 2, _E, _H), jnp.float32),
            pltpu.VMEM((_NB, _E, _E), jnp.bfloat16),
            pltpu.VMEM((_NB, 2, _H, _E), jnp.float32),
            pltpu.VMEM((_NB, 2, 1, _E), jnp.float32),
            pltpu.VMEM((_NB, 2, _E, _H), jnp.float32),
            pltpu.VMEM((_NB, 2, 1, _H), jnp.float32),
        ],
        compiler_params=pltpu.CompilerParams(
            dimension_semantics=("arbitrary",),
        ),
    )(x, wbig, bbig,
      w1_0, r(b1_0), w2_0, r(b2_0), w1_1, r(b1_1), w2_1, r(b2_1),
      m_w1_0, r(m_b1_0), m_w2_0, r(m_b2_0),
      m_w1_1, r(m_b1_1), m_w2_1, r(m_b2_1))
    return out.reshape(_S, _B, _H).transpose(1, 0, 2)


# speedup vs baseline: 1.0912x; 1.0912x over previous
import jax
import jax.numpy as jnp
from jax.experimental import pallas as pl
from jax.experimental.pallas import tpu as pltpu

_B, _S, _H = 4, 256, 128
_E = 2 * _H
_MAX_LR = 0.01
_NB = 4


def _sig(x):
    return jax.nn.sigmoid(x)


def _dot(a, b):
    return jax.lax.dot_general(a, b, (((1,), (0,)), ((), ())),
                               preferred_element_type=jnp.float32)


def _dot_t(a, b):
    return jax.lax.dot_general(a, b, (((1,), (1,)), ((), ())),
                               preferred_element_type=jnp.float32)


def _outer(a, b):
    return jax.lax.dot_general(a, b, (((0,), (0,)), ((), ())),
                               preferred_element_type=jnp.float32)


def _l2n(x):
    n = jnp.sqrt(jnp.sum(x * x, axis=-1, keepdims=True))
    return x / jnp.maximum(n, 1e-12)


def _mem_kernel(x_ref, wbig_ref, bbig_ref,
                w1_0_ref, b1_0_ref, w2_0_ref, b2_0_ref,
                w1_1_ref, b1_1_ref, w2_1_ref, b2_1_ref,
                mw1_0_ref, mb1_0_ref, mw2_0_ref, mb2_0_ref,
                mw1_1_ref, mb1_1_ref, mw2_1_ref, mb2_1_ref,
                o_ref,
                kr, vr, qr, scr, pq0, pk0,
                pw1b, pb1b, pw2, pb2,
                mw1, mb1, mw2, mb2):
    w1_refs = (w1_0_ref, w1_1_ref)
    b1_refs = (b1_0_ref, b1_1_ref)
    w2_refs = (w2_0_ref, w2_1_ref)
    b2_refs = (b2_0_ref, b2_1_ref)
    mw1_refs = (mw1_0_ref, mw1_1_ref)
    mb1_refs = (mb1_0_ref, mb1_1_ref)
    mw2_refs = (mw2_0_ref, mw2_1_ref)
    mb2_refs = (mb2_0_ref, mb2_1_ref)

    p00 = (w1_0_ref[...], b1_0_ref[...], w2_0_ref[...], b2_0_ref[...])
    p01 = (w1_1_ref[...], b1_1_ref[...], w2_1_ref[...], b2_1_ref[...])

    for b in range(_NB):
        xb = x_ref[b]
        proj = _dot(xb, wbig_ref[...]) + bbig_ref[...]
        qp = proj[:, 0:_H]
        kp = proj[:, _H:2 * _H]
        vp = proj[:, 2 * _H:3 * _H]
        gp = proj[:, 3 * _H:4 * _H]
        qn = _l2n(qp * _sig(qp))
        kn = _l2n(kp * _sig(kp))
        vn = vp * _sig(vp)
        sc = _sig(gp)
        cs = slice(b * _H, (b + 1) * _H)
        ce = slice(b * _E, (b + 1) * _E)
        qr[:, :, cs] = qn.reshape(_S, 1, _H)
        kr[0:_S, :, cs] = kn.reshape(_S, 1, _H)
        kr[_S:_S + 1, :, cs] = jnp.zeros((1, 1, _H), jnp.float32)
        vr[:, :, cs] = vn.reshape(_S, 1, _H)
        scr[:, :, cs] = sc.reshape(_S, 1, _H)
        pq0[:, :, ce] = _dot(qn, p00[0]).reshape(_S, 1, _E)
        pk0[0:_S, :, ce] = _dot(kn, p00[0]).reshape(_S, 1, _E)
        pk0[_S:_S + 1, :, ce] = jnp.zeros((1, 1, _E), jnp.float32)
        pw1b[b, 0] = w1_refs[1][...]
        pb1b[b, 0] = b1_refs[1][...]
        for d in range(2):
            pw2[b, d] = w2_refs[d][...]
            pb2[b, d] = b2_refs[d][...]
            mw1[b, d] = mw1_refs[d][...]
            mb1[b, d] = mb1_refs[d][...]
            mw2[b, d] = mw2_refs[d][...]
            mb2[b, d] = mb2_refs[d][...]

    krow0 = kr[0]
    init = []
    for b in range(_NB):
        cs = slice(b * _H, (b + 1) * _H)
        kt = krow0[:, cs]
        a1 = _dot(kt, p00[0]) + p00[1]
        sg1 = _sig(a1)
        s1 = a1 * sg1
        dsl1 = sg1 * (1.0 + a1 * (1.0 - sg1))
        h1 = kt + _dot(s1, p00[2]) + p00[3]
        a2 = _dot(h1, p01[0]) + p01[1]
        sg2 = _sig(a2)
        s2 = a2 * sg2
        dsl2 = sg2 * (1.0 + a2 * (1.0 - sg2))
        h2 = h1 + _dot(s2, p01[2]) + p01[3]
        init.append((s1, dsl1, h1, s2, dsl2, h2))

    def substep(t, carry):
        krow1 = kr[t + 1]
        vrow = vr[t]
        qrow = qr[t]
        srow = scr[t]
        krow = kr[t]
        pqrow = pq0[t]
        pkrow = pk0[t + 1]
        new_carry = []
        for b in range(_NB):
            cs = slice(b * _H, (b + 1) * _H)
            ce = slice(b * _E, (b + 1) * _E)
            s1, dsl1, h1, s2, dsl2, h2 = carry[b]
            kt = krow[:, cs]
            vt = vrow[:, cs]
            th = srow[:, b * _H:b * _H + 1] * _MAX_LR
            al = srow[:, b * _H + 1:b * _H + 2]
            et = srow[:, b * _H + 2:b * _H + 3]

            w1b = pw1b[b, 0]
            w2a = pw2[b, 0]
            w2b = pw2[b, 1]
            cm_w1a = mw1[b, 0]
            cm_b1a = mb1[b, 0]

            x2 = jnp.concatenate([qrow[:, cs], krow1[:, cs]], axis=0)
            p2 = jnp.concatenate([pqrow[:, ce], pkrow[:, ce]], axis=0)
            mx = _dot(x2, cm_w1a)
            rr = jnp.sum(x2 * kt, axis=1, keepdims=True)

            err = (2.0 / _H) * (h2 - vt)

            ds2 = _dot_t(err, w2b)
            da2 = ds2 * dsl2
            dh1 = err + _dot_t(da2, w1b)
            ds1 = _dot_t(dh1, w2a)
            da1 = ds1 * dsl1

            g_w2b = _outer(s2, err)
            g_w1b = _outer(h1, da2)
            g_w2a = _outer(s1, dh1)
            g_w1a = _outer(kt, da1)

            oma = 1.0 - al
            nm_w1a = et * cm_w1a - th * g_w1a
            nm_b1a = et * cm_b1a - th * da1
            nm_w2a = et * mw2[b, 0] - th * g_w2a
            nm_b2a = et * mb2[b, 0] - th * dh1
            nm_w1b = et * mw1[b, 1] - th * g_w1b
            nm_b1b = et * mb1[b, 1] - th * da2
            nm_w2b = et * mw2[b, 1] - th * g_w2b
            nm_b2b = et * mb2[b, 1] - th * err
            np_w2a = p00[2] * oma + nm_w2a
            np_b2a = p00[3] * oma + nm_b2a
            np_w1b = p01[0] * oma + nm_w1b
            np_b1b = p01[1] * oma + nm_b1b
            np_w2b = p01[2] * oma + nm_w2b
            np_b2b = p01[3] * oma + nm_b2b
            mw1[b, 0] = nm_w1a
            mb1[b, 0] = nm_b1a
            mw2[b, 0] = nm_w2a
            mb2[b, 0] = nm_b2a
            mw1[b, 1] = nm_w1b
            mb1[b, 1] = nm_b1b
            mw2[b, 1] = nm_w2b
            mb2[b, 1] = nm_b2b
            pw2[b, 0] = np_w2a
            pb2[b, 0] = np_b2a
            pw1b[b, 0] = np_w1b
            pb1b[b, 0] = np_b1b
            pw2[b, 1] = np_w2b
            pb2[b, 1] = np_b2b

            c1 = (oma * (p2 + p00[1]) + et * (mx + cm_b1a)
                  - th * ((rr + 1.0) * da1))
            sgc1 = _sig(c1)
            sc1 = c1 * sgc1
            u1 = x2 + _dot(sc1, np_w2a) + np_b2a
            c2 = _dot(u1, np_w1b) + np_b1b
            sgc2 = _sig(c2)
            sc2 = c2 * sgc2
            u2 = u1 + _dot(sc2, np_w2b) + np_b2b

            dslc1 = sgc1 * (1.0 + c1 * (1.0 - sgc1))
            dslc2 = sgc2 * (1.0 + c2 * (1.0 - sgc2))

            o_ref[pl.ds(t, 1), :, cs] = u2[0:1, :].reshape(1, 1, _H)
            new_carry.append((sc1[1:2, :], dslc1[1:2, :], u1[1:2, :],
                              sc2[1:2, :], dslc2[1:2, :], u2[1:2, :]))
        return tuple(new_carry)

    def step2(i, carry):
        t0 = i * 4
        for j in range(4):
            carry = substep(t0 + j, carry)
        return carry

    jax.lax.fori_loop(0, _S // 4, step2, tuple(init))


def kernel(x, Wq, Wk, Wv, w_lr, b_lr, w_fg, b_fg, w_mo, b_mo,
           w1_0, b1_0, w2_0, b2_0, w1_1, b1_1, w2_1, b2_1,
           m_w1_0, m_b1_0, m_w2_0, m_b2_0, m_w1_1, m_b1_1, m_w2_1, m_b2_1):
    lr3 = jnp.concatenate([w_lr, w_fg, w_mo], axis=1)
    lrpad = jnp.pad(lr3, ((0, 0), (0, _H - 3)))
    wbig = jnp.concatenate([Wq, Wk, Wv, lrpad], axis=1)
    bbig = jnp.zeros((1, 4 * _H), jnp.float32)
    bbig = bbig.at[0, 3 * _H].set(b_lr[0])
    bbig = bbig.at[0, 3 * _H + 1].set(b_fg[0])
    bbig = bbig.at[0, 3 * _H + 2].set(b_mo[0])

    r = lambda a: a.reshape(1, -1)

    def wspec(shape):
        return pl.BlockSpec(shape, lambda i: (0,) * len(shape))

    out = pl.pallas_call(
        _mem_kernel,
        grid=(1,),
        in_specs=[
            pl.BlockSpec((_NB, _S, _H), lambda i: (0, 0, 0)),
            wspec((_H, 4 * _H)), wspec((1, 4 * _H)),
            wspec((_H, _E)), wspec((1, _E)), wspec((_E, _H)), wspec((1, _H)),
            wspec((_H, _E)), wspec((1, _E)), wspec((_E, _H)), wspec((1, _H)),
            wspec((_H, _E)), wspec((1, _E)), wspec((_E, _H)), wspec((1, _H)),
            wspec((_H, _E)), wspec((1, _E)), wspec((_E, _H)), wspec((1, _H)),
        ],
        out_specs=pl.BlockSpec((_S, 1, _NB * _H), lambda i: (0, 0, 0)),
        out_shape=jax.ShapeDtypeStruct((_S, 1, _B * _H), jnp.float32),
        scratch_shapes=[
            pltpu.VMEM((_S + 1, 1, _NB * _H), jnp.float32),
            pltpu.VMEM((_S, 1, _NB * _H), jnp.float32),
            pltpu.VMEM((_S, 1, _NB * _H), jnp.float32),
            pltpu.VMEM((_S, 1, _NB * _H), jnp.float32),
            pltpu.VMEM((_S, 1, _NB * _E), jnp.float32),
            pltpu.VMEM((_S + 1, 1, _NB * _E), jnp.float32),
            pltpu.VMEM((_NB, 1, _H, _E), jnp.float32),
            pltpu.VMEM((_NB, 1, 1, _E), jnp.float32),
            pltpu.VMEM((_NB, 2, _E, _H), jnp.float32),
            pltpu.VMEM((_NB, 2, 1, _H), jnp.float32),
            pltpu.VMEM((_NB, 2, _H, _E), jnp.float32),
            pltpu.VMEM((_NB, 2, 1, _E), jnp.float32),
            pltpu.VMEM((_NB, 2, _E, _H), jnp.float32),
            pltpu.VMEM((_NB, 2, 1, _H), jnp.float32),
        ],
        compiler_params=pltpu.CompilerParams(
            dimension_semantics=("arbitrary",),
        ),
    )(x, wbig, bbig,
      w1_0, r(b1_0), w2_0, r(b2_0), w1_1, r(b1_1), w2_1, r(b2_1),
      m_w1_0, r(m_b1_0), m_w2_0, r(m_b2_0),
      m_w1_1, r(m_b1_1), m_w2_1, r(m_b2_1))
    return out.reshape(_S, _B, _H).transpose(1, 0, 2)


# speedup vs baseline: 1.1035x; 1.0112x over previous
import jax
import jax.numpy as jnp
from jax.experimental import pallas as pl
from jax.experimental.pallas import tpu as pltpu

_B, _S, _H = 4, 256, 128
_E = 2 * _H
_MAX_LR = 0.01
_NB = 4


def _sig(x):
    return jax.nn.sigmoid(x)


def _dot(a, b):
    return jax.lax.dot_general(a, b, (((1,), (0,)), ((), ())),
                               preferred_element_type=jnp.float32)


def _dot_t(a, b):
    return jax.lax.dot_general(a, b, (((1,), (1,)), ((), ())),
                               preferred_element_type=jnp.float32)


def _outer(a, b):
    return jax.lax.dot_general(a, b, (((0,), (0,)), ((), ())),
                               preferred_element_type=jnp.float32)


def _l2n(x):
    n = jnp.sqrt(jnp.sum(x * x, axis=-1, keepdims=True))
    return x / jnp.maximum(n, 1e-12)


def _mem_kernel(x_ref, wbig_ref, bbig_ref,
                w1_0_ref, b1_0_ref, w2_0_ref, b2_0_ref,
                w1_1_ref, b1_1_ref, w2_1_ref, b2_1_ref,
                mw1_0_ref, mb1_0_ref, mw2_0_ref, mb2_0_ref,
                mw1_1_ref, mb1_1_ref, mw2_1_ref, mb2_1_ref,
                o_ref,
                kr, vr, qr, scr, pq0, pk0,
                pw1b, pb1b, pw2, pb2,
                mw1, mb1, mw2, mb2):
    w1_refs = (w1_0_ref, w1_1_ref)
    b1_refs = (b1_0_ref, b1_1_ref)
    w2_refs = (w2_0_ref, w2_1_ref)
    b2_refs = (b2_0_ref, b2_1_ref)
    mw1_refs = (mw1_0_ref, mw1_1_ref)
    mb1_refs = (mb1_0_ref, mb1_1_ref)
    mw2_refs = (mw2_0_ref, mw2_1_ref)
    mb2_refs = (mb2_0_ref, mb2_1_ref)

    p00 = (w1_0_ref[...], b1_0_ref[...], w2_0_ref[...], b2_0_ref[...])
    p01 = (w1_1_ref[...], b1_1_ref[...], w2_1_ref[...], b2_1_ref[...])

    for b in range(_NB):
        xb = x_ref[b]
        proj = _dot(xb, wbig_ref[...]) + bbig_ref[...]
        qp = proj[:, 0:_H]
        kp = proj[:, _H:2 * _H]
        vp = proj[:, 2 * _H:3 * _H]
        gp = proj[:, 3 * _H:4 * _H]
        qn = _l2n(qp * _sig(qp))
        kn = _l2n(kp * _sig(kp))
        vn = vp * _sig(vp)
        sc = _sig(gp)
        cs = slice(b * _H, (b + 1) * _H)
        ce = slice(b * _E, (b + 1) * _E)
        qr[:, :, cs] = qn.reshape(_S, 1, _H)
        kr[0:_S, :, cs] = kn.reshape(_S, 1, _H)
        kr[_S:_S + 1, :, cs] = jnp.zeros((1, 1, _H), jnp.float32)
        vr[:, :, cs] = vn.reshape(_S, 1, _H)
        scr[:, :, cs] = sc.reshape(_S, 1, _H)
        pq0[:, :, ce] = _dot(qn, p00[0]).reshape(_S, 1, _E)
        pk0[0:_S, :, ce] = _dot(kn, p00[0]).reshape(_S, 1, _E)
        pk0[_S:_S + 1, :, ce] = jnp.zeros((1, 1, _E), jnp.float32)
        pw1b[b, 0] = w1_refs[1][...]
        pb1b[b, 0] = b1_refs[1][...]
        for d in range(2):
            pw2[b, d] = w2_refs[d][...]
            pb2[b, d] = b2_refs[d][...]
            mw1[b, d] = mw1_refs[d][...]
            mb1[b, d] = mb1_refs[d][...]
            mw2[b, d] = mw2_refs[d][...]
            mb2[b, d] = mb2_refs[d][...]

    krow0 = kr[0]
    init = []
    for b in range(_NB):
        cs = slice(b * _H, (b + 1) * _H)
        kt = krow0[:, cs]
        a1 = _dot(kt, p00[0]) + p00[1]
        sg1 = _sig(a1)
        s1 = a1 * sg1
        dsl1 = sg1 * (1.0 + a1 * (1.0 - sg1))
        h1 = kt + _dot(s1, p00[2]) + p00[3]
        a2 = _dot(h1, p01[0]) + p01[1]
        sg2 = _sig(a2)
        s2 = a2 * sg2
        dsl2 = sg2 * (1.0 + a2 * (1.0 - sg2))
        h2 = h1 + _dot(s2, p01[2]) + p01[3]
        init.append((s1, dsl1, h1, s2, dsl2, h2))

    def substep(t, carry):
        krow1 = kr[t + 1]
        vrow = vr[t]
        qrow = qr[t]
        srow = scr[t]
        krow = kr[t]
        pqrow = pq0[t]
        pkrow = pk0[t + 1]
        new_carry = []
        for b in range(_NB):
            cs = slice(b * _H, (b + 1) * _H)
            ce = slice(b * _E, (b + 1) * _E)
            s1, dsl1, h1, s2, dsl2, h2 = carry[b]
            kt = krow[:, cs]
            vt = vrow[:, cs]
            th = srow[:, b * _H:b * _H + 1] * _MAX_LR
            al = srow[:, b * _H + 1:b * _H + 2]
            et = srow[:, b * _H + 2:b * _H + 3]

            w1b = pw1b[b, 0]
            w2a = pw2[b, 0]
            w2b = pw2[b, 1]
            cm_w1a = mw1[b, 0]
            cm_b1a = mb1[b, 0]

            x2 = jnp.concatenate([qrow[:, cs], krow1[:, cs]], axis=0)
            p2 = jnp.concatenate([pqrow[:, ce], pkrow[:, ce]], axis=0)
            mx = _dot(x2, cm_w1a)
            rr = jnp.sum(x2 * kt, axis=1, keepdims=True)

            err = (2.0 / _H) * (h2 - vt)

            ds2 = _dot_t(err, w2b)
            da2 = ds2 * dsl2
            dh1 = err + _dot_t(da2, w1b)
            ds1 = _dot_t(dh1, w2a)
            da1 = ds1 * dsl1

            g_w2b = _outer(s2, err)
            g_w1b = _outer(h1, da2)
            g_w2a = _outer(s1, dh1)
            g_w1a = _outer(kt, da1)

            oma = 1.0 - al
            nm_w1a = et * cm_w1a - th * g_w1a
            nm_b1a = et * cm_b1a - th * da1
            nm_w2a = et * mw2[b, 0] - th * g_w2a
            nm_b2a = et * mb2[b, 0] - th * dh1
            nm_w1b = et * mw1[b, 1] - th * g_w1b
            nm_b1b = et * mb1[b, 1] - th * da2
            nm_w2b = et * mw2[b, 1] - th * g_w2b
            nm_b2b = et * mb2[b, 1] - th * err
            np_w2a = p00[2] * oma + nm_w2a
            np_b2a = p00[3] * oma + nm_b2a
            np_w1b = p01[0] * oma + nm_w1b
            np_b1b = p01[1] * oma + nm_b1b
            np_w2b = p01[2] * oma + nm_w2b
            np_b2b = p01[3] * oma + nm_b2b
            mw1[b, 0] = nm_w1a
            mb1[b, 0] = nm_b1a
            mw2[b, 0] = nm_w2a
            mb2[b, 0] = nm_b2a
            mw1[b, 1] = nm_w1b
            mb1[b, 1] = nm_b1b
            mw2[b, 1] = nm_w2b
            mb2[b, 1] = nm_b2b
            pw2[b, 0] = np_w2a
            pb2[b, 0] = np_b2a
            pw1b[b, 0] = np_w1b
            pb1b[b, 0] = np_b1b
            pw2[b, 1] = np_w2b
            pb2[b, 1] = np_b2b

            c1 = (oma * (p2 + p00[1]) + et * (mx + cm_b1a)
                  - th * ((rr + 1.0) * da1))
            sgc1 = _sig(c1)
            sc1 = c1 * sgc1
            u1 = x2 + _dot(sc1, np_w2a) + np_b2a
            c2 = _dot(u1, np_w1b) + np_b1b
            sgc2 = _sig(c2)
            sc2 = c2 * sgc2
            u2 = u1 + _dot(sc2, np_w2b) + np_b2b

            dslc1 = sgc1 * (1.0 + c1 * (1.0 - sgc1))
            dslc2 = sgc2 * (1.0 + c2 * (1.0 - sgc2))

            o_ref[pl.ds(t, 1), :, cs] = u2[0:1, :].reshape(1, 1, _H)
            new_carry.append((sc1[1:2, :], dslc1[1:2, :], u1[1:2, :],
                              sc2[1:2, :], dslc2[1:2, :], u2[1:2, :]))
        return tuple(new_carry)

    def step2(i, carry):
        t0 = i * 8
        for j in range(8):
            carry = substep(t0 + j, carry)
        return carry

    jax.lax.fori_loop(0, _S // 8, step2, tuple(init))


def kernel(x, Wq, Wk, Wv, w_lr, b_lr, w_fg, b_fg, w_mo, b_mo,
           w1_0, b1_0, w2_0, b2_0, w1_1, b1_1, w2_1, b2_1,
           m_w1_0, m_b1_0, m_w2_0, m_b2_0, m_w1_1, m_b1_1, m_w2_1, m_b2_1):
    lr3 = jnp.concatenate([w_lr, w_fg, w_mo], axis=1)
    lrpad = jnp.pad(lr3, ((0, 0), (0, _H - 3)))
    wbig = jnp.concatenate([Wq, Wk, Wv, lrpad], axis=1)
    bbig = jnp.zeros((1, 4 * _H), jnp.float32)
    bbig = bbig.at[0, 3 * _H].set(b_lr[0])
    bbig = bbig.at[0, 3 * _H + 1].set(b_fg[0])
    bbig = bbig.at[0, 3 * _H + 2].set(b_mo[0])

    r = lambda a: a.reshape(1, -1)

    def wspec(shape):
        return pl.BlockSpec(shape, lambda i: (0,) * len(shape))

    out = pl.pallas_call(
        _mem_kernel,
        grid=(1,),
        in_specs=[
            pl.BlockSpec((_NB, _S, _H), lambda i: (0, 0, 0)),
            wspec((_H, 4 * _H)), wspec((1, 4 * _H)),
            wspec((_H, _E)), wspec((1, _E)), wspec((_E, _H)), wspec((1, _H)),
            wspec((_H, _E)), wspec((1, _E)), wspec((_E, _H)), wspec((1, _H)),
            wspec((_H, _E)), wspec((1, _E)), wspec((_E, _H)), wspec((1, _H)),
            wspec((_H, _E)), wspec((1, _E)), wspec((_E, _H)), wspec((1, _H)),
        ],
        out_specs=pl.BlockSpec((_S, 1, _NB * _H), lambda i: (0, 0, 0)),
        out_shape=jax.ShapeDtypeStruct((_S, 1, _B * _H), jnp.float32),
        scratch_shapes=[
            pltpu.VMEM((_S + 1, 1, _NB * _H), jnp.float32),
            pltpu.VMEM((_S, 1, _NB * _H), jnp.float32),
            pltpu.VMEM((_S, 1, _NB * _H), jnp.float32),
            pltpu.VMEM((_S, 1, _NB * _H), jnp.float32),
            pltpu.VMEM((_S, 1, _NB * _E), jnp.float32),
            pltpu.VMEM((_S + 1, 1, _NB * _E), jnp.float32),
            pltpu.VMEM((_NB, 1, _H, _E), jnp.float32),
            pltpu.VMEM((_NB, 1, 1, _E), jnp.float32),
            pltpu.VMEM((_NB, 2, _E, _H), jnp.float32),
            pltpu.VMEM((_NB, 2, 1, _H), jnp.float32),
            pltpu.VMEM((_NB, 2, _H, _E), jnp.float32),
            pltpu.VMEM((_NB, 2, 1, _E), jnp.float32),
            pltpu.VMEM((_NB, 2, _E, _H), jnp.float32),
            pltpu.VMEM((_NB, 2, 1, _H), jnp.float32),
        ],
        compiler_params=pltpu.CompilerParams(
            dimension_semantics=("arbitrary",),
        ),
    )(x, wbig, bbig,
      w1_0, r(b1_0), w2_0, r(b2_0), w1_1, r(b1_1), w2_1, r(b2_1),
      m_w1_0, r(m_b1_0), m_w2_0, r(m_b2_0),
      m_w1_1, r(m_b1_1), m_w2_1, r(m_b2_1))
    return out.reshape(_S, _B, _H).transpose(1, 0, 2)
